# baseline (speedup 1.0000x reference)
import functools

import jax
import jax.numpy as jnp
from jax import lax
from jax.experimental import pallas as pl
from jax.experimental.pallas import tpu as pltpu
from jax.experimental.pallas import tpu_sc as plsc

D = 768
PAD_IDX = 1
EPS = 1e-5

NC = 2
NS = 16
NW = NC * NS
CHUNK = 32
NBUF = 4
DEPTH = 3
NSLAB = 4


def _sc_gather(ids2d, table):
    n_chunks = ids2d.shape[0]
    ch_per_w = n_chunks // NW
    n_rows = n_chunks * CHUNK
    mesh = plsc.VectorSubcoreMesh(core_axis_name="c", subcore_axis_name="s")

    @functools.partial(
        pl.kernel,
        mesh=mesh,
        out_type=jax.ShapeDtypeStruct((n_rows, D), jnp.float32),
        scratch_types=[
            pltpu.VMEM((ch_per_w, CHUNK), jnp.int32),
            pltpu.VMEM((CHUNK, D), jnp.float32),
            pltpu.VMEM((CHUNK, D), jnp.float32),
            pltpu.VMEM((CHUNK, D), jnp.float32),
            pltpu.VMEM((CHUNK, D), jnp.float32),
            pltpu.SemaphoreType.DMA,
            pltpu.SemaphoreType.DMA,
            pltpu.SemaphoreType.DMA,
            pltpu.SemaphoreType.DMA,
            pltpu.SemaphoreType.DMA,
            pltpu.SemaphoreType.DMA,
            pltpu.SemaphoreType.DMA,
            pltpu.SemaphoreType.DMA,
        ],
    )
    def k(ids_hbm, tab_hbm, out_hbm, idx_v, b0, b1, b2, b3,
          g0, g1, g2, g3, o0, o1, o2, o3):
        wid = lax.axis_index("s") * NC + lax.axis_index("c")
        base = wid * ch_per_w
        pltpu.sync_copy(ids_hbm.at[pl.ds(base, ch_per_w)], idx_v)
        bufs = (b0, b1, b2, b3)
        gsem = (g0, g1, g2, g3)
        osem = (o0, o1, o2, o3)
        depth = min(DEPTH, ch_per_w)
        gcopy = {}
        ocopy = {}
        out_waited = set()
        for j in range(depth):
            gcopy[j] = pltpu.async_copy(
                tab_hbm.at[idx_v.at[j]], bufs[j % NBUF], gsem[j % NBUF])
        for c in range(ch_per_w):
            cur = c % NBUF
            gcopy[c].wait()
            ocopy[c] = pltpu.async_copy(
                bufs[cur], out_hbm.at[pl.ds((base + c) * CHUNK, CHUNK)],
                osem[cur])
            nc_ = c + depth
            if nc_ < ch_per_w:
                nb = nc_ % NBUF
                prev_out = nc_ - NBUF
                if prev_out >= 0:
                    ocopy[prev_out].wait()
                    out_waited.add(prev_out)
                gcopy[nc_] = pltpu.async_copy(
                    tab_hbm.at[idx_v.at[nc_]], bufs[nb], gsem[nb])
        for c in range(ch_per_w):
            if c not in out_waited:
                ocopy[c].wait()

    return k(ids2d, table)


def _ln_body(g_ref, p_ref, t_ref, ga_ref, be_ref, *rest):
    o_ref = rest[-1]
    x = g_ref[...] + p_ref[...] + t_ref[...]
    mean = jnp.mean(x, axis=-1, keepdims=True)
    xc = x - mean
    var = jnp.mean(xc * xc, axis=-1, keepdims=True)
    o_ref[...] = xc * lax.rsqrt(var + EPS) * ga_ref[...] + be_ref[...]


def _tc_ln_slab(g_k, pos_k, type0, gamma, beta, out_prev, k, n_b, s_total):
    blk = 512
    sbk = pos_k.shape[0] // blk
    sb_total = s_total // blk
    n_rows = n_b * s_total

    base_specs = [
        pl.BlockSpec((blk, D), lambda s, b: (b * sbk + s, 0)),
        pl.BlockSpec((blk, D), lambda s, b: (s, 0)),
        pl.BlockSpec((1, D), lambda s, b: (0, 0)),
        pl.BlockSpec((1, D), lambda s, b: (0, 0)),
        pl.BlockSpec((1, D), lambda s, b: (0, 0)),
    ]
    out_spec = pl.BlockSpec(
        (blk, D), lambda s, b: (b * sb_total + k * sbk + s, 0))
    out_shape = jax.ShapeDtypeStruct((n_rows, D), jnp.float32)
    args = [g_k, pos_k, type0, gamma, beta]
    if out_prev is None:
        return pl.pallas_call(
            _ln_body, grid=(sbk, n_b), in_specs=base_specs,
            out_specs=out_spec, out_shape=out_shape,
        )(*args)
    return pl.pallas_call(
        _ln_body, grid=(sbk, n_b),
        in_specs=base_specs + [pl.BlockSpec(memory_space=pl.ANY)],
        out_specs=out_spec, out_shape=out_shape,
        input_output_aliases={5: 0},
    )(*args, out_prev)


def kernel(input_ids, word_embeddings, position_embeddings,
           token_type_embeddings, ln_gamma, ln_beta):
    b_sz, s_len = input_ids.shape
    slab_s = s_len // NSLAB
    ids32 = input_ids.astype(jnp.int32)
    pos_sl = position_embeddings[PAD_IDX + 1:PAD_IDX + 1 + s_len]
    type0 = token_type_embeddings[:1]
    gamma = ln_gamma.reshape(1, D)
    beta = ln_beta.reshape(1, D)
    gs = []
    for k in range(NSLAB):
        ids_k = ids32[:, k * slab_s:(k + 1) * slab_s].reshape(-1, CHUNK)
        gs.append(_sc_gather(ids_k, word_embeddings))
    out = None
    for k in range(NSLAB):
        out = _tc_ln_slab(
            gs[k], pos_sl[k * slab_s:(k + 1) * slab_s], type0, gamma, beta,
            out, k, b_sz, s_len)
    return out.reshape(b_sz, s_len, D)


# speedup vs baseline: 1.0505x; 1.0505x over previous
import functools

import jax
import jax.numpy as jnp
from jax import lax
from jax.experimental import pallas as pl
from jax.experimental.pallas import tpu as pltpu
from jax.experimental.pallas import tpu_sc as plsc

D = 768
PAD_IDX = 1
EPS = 1e-5

NC = 2
NS = 16
NW = NC * NS
CHUNK = 32
NBUF = 4
DEPTH = 3
NSLAB = 4


def _sc_gather(ids2d, table):
    n_chunks = ids2d.shape[0]
    ch_per_w = n_chunks // NW
    n_rows = n_chunks * CHUNK
    mesh = plsc.VectorSubcoreMesh(core_axis_name="c", subcore_axis_name="s")

    @functools.partial(
        pl.kernel,
        mesh=mesh,
        out_type=jax.ShapeDtypeStruct((n_rows, D), jnp.float32),
        scratch_types=[
            pltpu.VMEM((ch_per_w, CHUNK), jnp.int32),
            pltpu.VMEM((CHUNK, D), jnp.float32),
            pltpu.VMEM((CHUNK, D), jnp.float32),
            pltpu.VMEM((CHUNK, D), jnp.float32),
            pltpu.VMEM((CHUNK, D), jnp.float32),
            pltpu.SemaphoreType.DMA,
            pltpu.SemaphoreType.DMA,
            pltpu.SemaphoreType.DMA,
            pltpu.SemaphoreType.DMA,
            pltpu.SemaphoreType.DMA,
            pltpu.SemaphoreType.DMA,
            pltpu.SemaphoreType.DMA,
            pltpu.SemaphoreType.DMA,
        ],
    )
    def k(ids_hbm, tab_hbm, out_hbm, idx_v, b0, b1, b2, b3,
          g0, g1, g2, g3, o0, o1, o2, o3):
        wid = lax.axis_index("s") * NC + lax.axis_index("c")
        base = wid * ch_per_w
        pltpu.sync_copy(ids_hbm.at[pl.ds(base, ch_per_w)], idx_v)
        bufs = (b0, b1, b2, b3)
        gsem = (g0, g1, g2, g3)
        osem = (o0, o1, o2, o3)
        depth = min(DEPTH, ch_per_w)
        gcopy = {}
        ocopy = {}
        out_waited = set()
        for j in range(depth):
            gcopy[j] = pltpu.async_copy(
                tab_hbm.at[idx_v.at[j]], bufs[j % NBUF], gsem[j % NBUF])
        for c in range(ch_per_w):
            cur = c % NBUF
            gcopy[c].wait()
            ocopy[c] = pltpu.async_copy(
                bufs[cur], out_hbm.at[pl.ds((base + c) * CHUNK, CHUNK)],
                osem[cur])
            nc_ = c + depth
            if nc_ < ch_per_w:
                nb = nc_ % NBUF
                prev_out = nc_ - NBUF
                if prev_out >= 0:
                    ocopy[prev_out].wait()
                    out_waited.add(prev_out)
                gcopy[nc_] = pltpu.async_copy(
                    tab_hbm.at[idx_v.at[nc_]], bufs[nb], gsem[nb])
        for c in range(ch_per_w):
            if c not in out_waited:
                ocopy[c].wait()

    return k(ids2d, table)


def _ln_body(g_ref, p_ref, t_ref, ga_ref, be_ref, *rest):
    o_ref = rest[-1]
    x = g_ref[...] + p_ref[...] + t_ref[...]
    mean = jnp.mean(x, axis=-1, keepdims=True)
    xc = x - mean
    var = jnp.mean(xc * xc, axis=-1, keepdims=True)
    o_ref[...] = xc * lax.rsqrt(var + EPS) * ga_ref[...] + be_ref[...]


def _tc_ln_slab(g_k, pos_k, type0, gamma, beta, out_prev, k, n_b, s_total):
    blk = 1024
    sbk = pos_k.shape[0] // blk
    sb_total = s_total // blk
    n_rows = n_b * s_total

    base_specs = [
        pl.BlockSpec((blk, D), lambda s, b: (b * sbk + s, 0)),
        pl.BlockSpec((blk, D), lambda s, b: (s, 0)),
        pl.BlockSpec((1, D), lambda s, b: (0, 0)),
        pl.BlockSpec((1, D), lambda s, b: (0, 0)),
        pl.BlockSpec((1, D), lambda s, b: (0, 0)),
    ]
    out_spec = pl.BlockSpec(
        (blk, D), lambda s, b: (b * sb_total + k * sbk + s, 0))
    out_shape = jax.ShapeDtypeStruct((n_rows, D), jnp.float32)
    args = [g_k, pos_k, type0, gamma, beta]
    if out_prev is None:
        return pl.pallas_call(
            _ln_body, grid=(sbk, n_b), in_specs=base_specs,
            out_specs=out_spec, out_shape=out_shape,
        )(*args)
    return pl.pallas_call(
        _ln_body, grid=(sbk, n_b),
        in_specs=base_specs + [pl.BlockSpec(memory_space=pl.ANY)],
        out_specs=out_spec, out_shape=out_shape,
        input_output_aliases={5: 0},
    )(*args, out_prev)


def kernel(input_ids, word_embeddings, position_embeddings,
           token_type_embeddings, ln_gamma, ln_beta):
    b_sz, s_len = input_ids.shape
    slab_s = s_len // NSLAB
    ids32 = input_ids.astype(jnp.int32)
    pos_sl = position_embeddings[PAD_IDX + 1:PAD_IDX + 1 + s_len]
    type0 = token_type_embeddings[:1]
    gamma = ln_gamma.reshape(1, D)
    beta = ln_beta.reshape(1, D)
    gs = []
    for k in range(NSLAB):
        ids_k = ids32[:, k * slab_s:(k + 1) * slab_s].reshape(-1, CHUNK)
        gs.append(_sc_gather(ids_k, word_embeddings))
    out = None
    for k in range(NSLAB):
        out = _tc_ln_slab(
            gs[k], pos_sl[k * slab_s:(k + 1) * slab_s], type0, gamma, beta,
            out, k, b_sz, s_len)
    return out.reshape(b_sz, s_len, D)


# speedup vs baseline: 1.0656x; 1.0143x over previous
import functools

import jax
import jax.numpy as jnp
from jax import lax
from jax.experimental import pallas as pl
from jax.experimental.pallas import tpu as pltpu
from jax.experimental.pallas import tpu_sc as plsc

D = 768
PAD_IDX = 1
EPS = 1e-5

NC = 2
NS = 16
NW = NC * NS
CHUNK = 32
NBUF = 4
DEPTH = 3
NSLAB = 4


def _sc_gather(ids2d, table):
    n_chunks = ids2d.shape[0]
    ch_per_w = n_chunks // NW
    n_rows = n_chunks * CHUNK
    mesh = plsc.VectorSubcoreMesh(core_axis_name="c", subcore_axis_name="s")

    @functools.partial(
        pl.kernel,
        mesh=mesh,
        out_type=jax.ShapeDtypeStruct((n_rows, D), jnp.float32),
        scratch_types=[
            pltpu.VMEM((ch_per_w, CHUNK), jnp.int32),
            pltpu.VMEM((CHUNK, D), jnp.float32),
            pltpu.VMEM((CHUNK, D), jnp.float32),
            pltpu.VMEM((CHUNK, D), jnp.float32),
            pltpu.VMEM((CHUNK, D), jnp.float32),
            pltpu.SemaphoreType.DMA,
            pltpu.SemaphoreType.DMA,
            pltpu.SemaphoreType.DMA,
            pltpu.SemaphoreType.DMA,
            pltpu.SemaphoreType.DMA,
            pltpu.SemaphoreType.DMA,
            pltpu.SemaphoreType.DMA,
            pltpu.SemaphoreType.DMA,
        ],
    )
    def k(ids_hbm, tab_hbm, out_hbm, idx_v, b0, b1, b2, b3,
          g0, g1, g2, g3, o0, o1, o2, o3):
        wid = lax.axis_index("s") * NC + lax.axis_index("c")
        base = wid * ch_per_w
        pltpu.sync_copy(ids_hbm.at[pl.ds(base, ch_per_w)], idx_v)
        bufs = (b0, b1, b2, b3)
        gsem = (g0, g1, g2, g3)
        osem = (o0, o1, o2, o3)
        depth = min(DEPTH, ch_per_w)
        gcopy = {}
        ocopy = {}
        out_waited = set()
        for j in range(depth):
            gcopy[j] = pltpu.async_copy(
                tab_hbm.at[idx_v.at[j]], bufs[j % NBUF], gsem[j % NBUF])
        for c in range(ch_per_w):
            cur = c % NBUF
            gcopy[c].wait()
            ocopy[c] = pltpu.async_copy(
                bufs[cur], out_hbm.at[pl.ds((base + c) * CHUNK, CHUNK)],
                osem[cur])
            nc_ = c + depth
            if nc_ < ch_per_w:
                nb = nc_ % NBUF
                prev_out = nc_ - NBUF
                if prev_out >= 0:
                    ocopy[prev_out].wait()
                    out_waited.add(prev_out)
                gcopy[nc_] = pltpu.async_copy(
                    tab_hbm.at[idx_v.at[nc_]], bufs[nb], gsem[nb])
        for c in range(ch_per_w):
            if c not in out_waited:
                ocopy[c].wait()

    return k(ids2d, table)


def _ln_body(g_ref, p_ref, t_ref, ga_ref, be_ref, *rest):
    o_ref = rest[-1]
    x = g_ref[...] + p_ref[...] + t_ref[...]
    mean = jnp.mean(x, axis=-1, keepdims=True)
    xc = x - mean
    var = jnp.mean(xc * xc, axis=-1, keepdims=True)
    o_ref[...] = xc * lax.rsqrt(var + EPS) * ga_ref[...] + be_ref[...]


def _tc_ln_slab(g_k, pos_k, type0, gamma, beta, out_prev, k, n_b, s_total):
    blk = 2048
    sbk = pos_k.shape[0] // blk
    sb_total = s_total // blk
    n_rows = n_b * s_total

    base_specs = [
        pl.BlockSpec((blk, D), lambda s, b: (b * sbk + s, 0)),
        pl.BlockSpec((blk, D), lambda s, b: (s, 0)),
        pl.BlockSpec((1, D), lambda s, b: (0, 0)),
        pl.BlockSpec((1, D), lambda s, b: (0, 0)),
        pl.BlockSpec((1, D), lambda s, b: (0, 0)),
    ]
    out_spec = pl.BlockSpec(
        (blk, D), lambda s, b: (b * sb_total + k * sbk + s, 0))
    out_shape = jax.ShapeDtypeStruct((n_rows, D), jnp.float32)
    args = [g_k, pos_k, type0, gamma, beta]
    if out_prev is None:
        return pl.pallas_call(
            _ln_body, grid=(sbk, n_b), in_specs=base_specs,
            out_specs=out_spec, out_shape=out_shape,
        )(*args)
    return pl.pallas_call(
        _ln_body, grid=(sbk, n_b),
        in_specs=base_specs + [pl.BlockSpec(memory_space=pl.ANY)],
        out_specs=out_spec, out_shape=out_shape,
        input_output_aliases={5: 0},
    )(*args, out_prev)


def kernel(input_ids, word_embeddings, position_embeddings,
           token_type_embeddings, ln_gamma, ln_beta):
    b_sz, s_len = input_ids.shape
    slab_s = s_len // NSLAB
    ids32 = input_ids.astype(jnp.int32)
    pos_sl = position_embeddings[PAD_IDX + 1:PAD_IDX + 1 + s_len]
    type0 = token_type_embeddings[:1]
    gamma = ln_gamma.reshape(1, D)
    beta = ln_beta.reshape(1, D)
    gs = []
    for k in range(NSLAB):
        ids_k = ids32[:, k * slab_s:(k + 1) * slab_s].reshape(-1, CHUNK)
        gs.append(_sc_gather(ids_k, word_embeddings))
    out = None
    for k in range(NSLAB):
        out = _tc_ln_slab(
            gs[k], pos_sl[k * slab_s:(k + 1) * slab_s], type0, gamma, beta,
            out, k, b_sz, s_len)
    return out.reshape(b_sz, s_len, D)
